# baseline (speedup 1.0000x reference)
import functools

import jax
import jax.numpy as jnp
from jax import lax
from jax.experimental import pallas as pl
from jax.experimental.pallas import tpu as pltpu
from jax.experimental.pallas import tpu_sc as plsc

N = 10000
E = 320000
G = 64
NC, NS, LANES = 2, 16, 16
NW = NC * NS
GROUP = 128
DGPW = 160
GROUP64 = 64
GPW = 320
CHG = 32
NBUF = 4
E_PAD = NW * GPW * GROUP64
NPAD = 10240
ROWS_PT = NPAD // NS
NDPAD = 10240
DEG_PT = NDPAD // NS

_f32 = jnp.float32

_MESH = plsc.VectorSubcoreMesh(
    core_axis_name="c", subcore_axis_name="s", num_cores=NC, num_subcores=NS)


def _worker_id():
    c = lax.axis_index("c")
    s = lax.axis_index("s")
    return c, s, c * NS + s


def _sc_deg_body(dst_hbm, ew_hbm, zdeg_hbm, out_hbm, dst_all, ew_all,
                 deg_sh, sem):
    c, s, _ = _worker_id()

    @pl.when(c == 0)
    def _work():
        pltpu.sync_copy(zdeg_hbm.at[pl.ds(s * DEG_PT, DEG_PT)],
                        deg_sh.at[pl.ds(s * DEG_PT, DEG_PT)])
        pltpu.sync_copy(dst_hbm.at[pl.ds(s * DGPW, DGPW), :], dst_all)
        pltpu.sync_copy(ew_hbm.at[pl.ds(s * DGPW, DGPW), :], ew_all)
        plsc.subcore_barrier()

        def chunk(t, _):
            for b in range(16):
                g = t * 16 + b
                pltpu.async_copy(ew_all.at[g], deg_sh.at[dst_all.at[g]], sem,
                                 add=True)
            for b in range(16):
                g = t * 16 + b
                pltpu.make_async_copy(ew_all.at[g], deg_sh.at[dst_all.at[g]],
                                      sem).wait()
            return 0

        lax.fori_loop(0, DGPW // 16, chunk, 0)
        plsc.subcore_barrier()
        pltpu.sync_copy(deg_sh.at[pl.ds(s * DEG_PT, DEG_PT)],
                        out_hbm.at[pl.ds(s * DEG_PT, DEG_PT)])


def _sc_deg(dst2d, ew2d, zdeg):
    k = pl.kernel(
        _sc_deg_body,
        out_type=jax.ShapeDtypeStruct((NDPAD,), _f32),
        mesh=_MESH,
        scratch_types=[
            pltpu.VMEM((DGPW, GROUP), jnp.int32),
            pltpu.VMEM((DGPW, GROUP), _f32),
            pltpu.VMEM_SHARED((NDPAD,), _f32),
            pltpu.SemaphoreType.DMA,
        ],
    )
    return k(dst2d, ew2d, zdeg)


def _sc_scatter_body(F, src_hbm, dst_hbm, ew_hbm, xs_hbm, zacc_hbm, out_hbm,
                     src_all, dst_all, ew_all, rows0, rows1, rows2, rows3,
                     acc_sh, gsem0, gsem1, gsem2, gsem3,
                     ssem0, ssem1, ssem2, ssem3):
    c, s, _ = _worker_id()
    nfb = F // LANES
    rows = (rows0, rows1, rows2, rows3)
    gsem = (gsem0, gsem1, gsem2, gsem3)
    ssem = (ssem0, ssem1, ssem2, ssem3)

    def chunk_loop(t, _):
        base = s * GPW + t * CHG
        pltpu.sync_copy(src_hbm.at[pl.ds(base, CHG), :], src_all)
        pltpu.sync_copy(dst_hbm.at[pl.ds(base, CHG), :], dst_all)
        pltpu.sync_copy(ew_hbm.at[pl.ds(base, CHG), :], ew_all)
        pltpu.async_copy(xs_hbm.at[src_all.at[0]], rows[0], gsem[0])
        pltpu.async_copy(xs_hbm.at[src_all.at[1]], rows[1], gsem[1])

        def outer(u, _):
            for b in range(NBUF):
                g = u * NBUF + b
                rows_b = rows[b]
                pltpu.make_async_copy(xs_hbm.at[src_all.at[g]], rows_b,
                                      gsem[b]).wait()

                for jg in range(GROUP64 // LANES):
                    wv = ew_all[g, pl.ds(jg * LANES, LANES)]
                    for l in range(LANES):
                        w = jnp.full((LANES,), wv[l], dtype=_f32)
                        j = jg * LANES + l
                        for f in range(nfb):
                            rows_b[j, pl.ds(f * LANES, LANES)] = (
                                rows_b[j, pl.ds(f * LANES, LANES)] * w)
                pltpu.async_copy(rows_b, acc_sh.at[dst_all.at[g]], ssem[b],
                                 add=True)

                bn = (b + 2) % NBUF
                rows_n = rows[bn]

                @pl.when(g + 2 < CHG)
                def _next():
                    @pl.when(g >= 2)
                    def _drain():
                        pltpu.make_async_copy(
                            rows_n, acc_sh.at[dst_all.at[g]], ssem[bn]).wait()

                    pltpu.async_copy(xs_hbm.at[src_all.at[g + 2]], rows_n,
                                     gsem[bn])
            return 0

        lax.fori_loop(0, CHG // NBUF, outer, 0)
        for b in range(NBUF):
            pltpu.make_async_copy(rows[b], acc_sh.at[dst_all.at[0]],
                                  ssem[b]).wait()
        return 0

    @pl.when(c == 0)
    def _work():
        pltpu.sync_copy(zacc_hbm.at[pl.ds(s * ROWS_PT, ROWS_PT), :],
                        acc_sh.at[pl.ds(s * ROWS_PT, ROWS_PT), :])
        plsc.subcore_barrier()
        lax.fori_loop(0, GPW // CHG, chunk_loop, 0)
        plsc.subcore_barrier()
        pltpu.sync_copy(acc_sh.at[pl.ds(s * ROWS_PT, ROWS_PT), :],
                        out_hbm.at[pl.ds(s * ROWS_PT, ROWS_PT), :])


def _sc_scatter(F, src2d, dst2d, ew2d, xs, zacc):
    k = pl.kernel(
        functools.partial(_sc_scatter_body, F),
        out_type=jax.ShapeDtypeStruct((NPAD, F), _f32),
        mesh=_MESH,
        scratch_types=(
            [pltpu.VMEM((CHG, GROUP64), jnp.int32),
             pltpu.VMEM((CHG, GROUP64), jnp.int32),
             pltpu.VMEM((CHG, GROUP64), _f32)]
            + [pltpu.VMEM((GROUP64, F), _f32) for _ in range(NBUF)]
            + [pltpu.VMEM_SHARED((NPAD, F), _f32)]
            + [pltpu.SemaphoreType.DMA for _ in range(2 * NBUF)]
        ),
        compiler_params=pltpu.CompilerParams(use_tc_tiling_on_sc=False),
    )
    return k(src2d, dst2d, ew2d, xs, zacc)


_BLK = 1000
_NBLK = N // _BLK


def _tc_prep1_body(x_ref, degt_ref, w1_ref, xs_ref, dinv_ref):
    deg = jnp.sum(degt_ref[...], axis=1, keepdims=True) + 1.0
    dinv = lax.rsqrt(deg)
    dinv_ref[...] = dinv
    xs_ref[...] = jnp.dot(x_ref[...], w1_ref[...],
                          preferred_element_type=_f32) * dinv


def _tc_prep1(x, degt, W1):
    return pl.pallas_call(
        _tc_prep1_body,
        grid=(_NBLK,),
        in_specs=[
            pl.BlockSpec((_BLK, 128), lambda i: (i, 0)),
            pl.BlockSpec((_BLK, 1), lambda i: (i, 0)),
            pl.BlockSpec((128, 128), lambda i: (0, 0)),
        ],
        out_specs=[
            pl.BlockSpec((_BLK, 128), lambda i: (i, 0)),
            pl.BlockSpec((_BLK, 1), lambda i: (i, 0)),
        ],
        out_shape=[
            jax.ShapeDtypeStruct((N, 128), _f32),
            jax.ShapeDtypeStruct((N, 1), _f32),
        ],
    )(x, degt, W1)


def _tc_mid_body(acc_ref, xs1_ref, dinv_ref, w2_ref, b1_ref, xs2_ref):
    dinv = dinv_ref[...]
    agg = acc_ref[...] + xs1_ref[...]
    h = jax.nn.relu(dinv * agg + b1_ref[...])
    xs2_ref[...] = jnp.dot(h, w2_ref[...], preferred_element_type=_f32) * dinv


def _tc_mid(acc1, xs1, dinv, W2, b1):
    return pl.pallas_call(
        _tc_mid_body,
        grid=(_NBLK,),
        in_specs=[
            pl.BlockSpec((_BLK, 128), lambda i: (i, 0)),
            pl.BlockSpec((_BLK, 128), lambda i: (i, 0)),
            pl.BlockSpec((_BLK, 1), lambda i: (i, 0)),
            pl.BlockSpec((128, 64), lambda i: (0, 0)),
            pl.BlockSpec((1, 128), lambda i: (0, 0)),
        ],
        out_specs=pl.BlockSpec((_BLK, 64), lambda i: (i, 0)),
        out_shape=jax.ShapeDtypeStruct((N, 64), _f32),
    )(acc1, xs1, dinv, W2, b1)


def _tc_final_body(acc_ref, xs2_ref, dinv_ref, b2_ref, batch_ref,
                   fc1w_ref, fc1b_ref, fc2w_ref, fc2b_ref, y_ref,
                   pool_acc, cnt_acc):
    i = pl.program_id(0)
    agg = acc_ref[...] + xs2_ref[...]
    x1 = jax.nn.relu(dinv_ref[...] * agg + b2_ref[...])
    gids = lax.broadcasted_iota(jnp.int32, (_BLK, G), 1)
    onehot = jnp.where(batch_ref[...] == gids, 1.0, 0.0)
    pool_blk = lax.dot_general(onehot, x1, (((0,), (0,)), ((), ())),
                               preferred_element_type=_f32)
    cnt_blk = lax.dot_general(onehot, jnp.ones((_BLK, 1), _f32),
                              (((0,), (0,)), ((), ())),
                              preferred_element_type=_f32)

    @pl.when(i == 0)
    def _init():
        pool_acc[...] = jnp.zeros_like(pool_acc)
        cnt_acc[...] = jnp.zeros_like(cnt_acc)

    pool_acc[...] += pool_blk
    cnt_acc[...] += cnt_blk

    @pl.when(i == _NBLK - 1)
    def _fin():
        mean = pool_acc[...] / jnp.maximum(cnt_acc[...], 1.0)
        h2 = jax.nn.relu(jnp.dot(mean, fc1w_ref[...],
                                 preferred_element_type=_f32) + fc1b_ref[...])
        logits = jnp.dot(h2, fc2w_ref[...],
                         preferred_element_type=_f32) + fc2b_ref[...]
        m = jnp.max(logits, axis=1, keepdims=True)
        lse = m + jnp.log(jnp.sum(jnp.exp(logits - m), axis=1, keepdims=True))
        y_ref[...] = logits - lse


def _tc_final(acc2, xs2, dinv, b2, batchf, fc1_w, fc1_b, fc2_w, fc2_b):
    return pl.pallas_call(
        _tc_final_body,
        grid=(_NBLK,),
        in_specs=[
            pl.BlockSpec((_BLK, 64), lambda i: (i, 0)),
            pl.BlockSpec((_BLK, 64), lambda i: (i, 0)),
            pl.BlockSpec((_BLK, 1), lambda i: (i, 0)),
            pl.BlockSpec((1, 64), lambda i: (0, 0)),
            pl.BlockSpec((_BLK, 1), lambda i: (i, 0)),
            pl.BlockSpec((64, 128), lambda i: (0, 0)),
            pl.BlockSpec((1, 128), lambda i: (0, 0)),
            pl.BlockSpec((128, 2), lambda i: (0, 0)),
            pl.BlockSpec((1, 2), lambda i: (0, 0)),
        ],
        out_specs=pl.BlockSpec((G, 2), lambda i: (0, 0)),
        out_shape=jax.ShapeDtypeStruct((G, 2), _f32),
        scratch_shapes=[
            pltpu.VMEM((G, 64), _f32),
            pltpu.VMEM((G, 1), _f32),
        ],
        compiler_params=pltpu.CompilerParams(
            dimension_semantics=("arbitrary",)),
    )(acc2, xs2, dinv, b2, batchf, fc1_w, fc1_b, fc2_w, fc2_b)


def kernel(x, edge_index, edge_attr, batch, W1, b1, W2, b2,
           fc1_w, fc1_b, fc2_w, fc2_b):
    ew = jnp.squeeze(edge_attr).astype(_f32)
    src = edge_index[0]
    dst = edge_index[1]

    pad = E_PAD - E
    srcp = jnp.concatenate([src, jnp.zeros((pad,), src.dtype)])
    dstp = jnp.concatenate([dst, jnp.zeros((pad,), dst.dtype)])
    ewp = jnp.concatenate([ew, jnp.zeros((pad,), _f32)])
    dst2d = dstp.reshape(-1, GROUP)
    ew2d = ewp.reshape(-1, GROUP)
    src64 = srcp.reshape(-1, GROUP64)
    dst64 = dstp.reshape(-1, GROUP64)
    ew64 = ewp.reshape(-1, GROUP64)

    zdeg = jnp.zeros((NDPAD,), _f32)
    zacc128 = jnp.zeros((NPAD, 128), _f32)
    zacc64 = jnp.zeros((NPAD, 64), _f32)

    degp = _sc_deg(dst2d, ew2d, zdeg)
    degt = degp[:N].reshape(N, 1)

    xs1, dinv = _tc_prep1(x, degt, W1)
    acc1 = _sc_scatter(128, src64, dst64, ew64, xs1, zacc128)[:N]
    xs2 = _tc_mid(acc1, xs1, dinv, W2, b1.reshape(1, 128))
    acc2 = _sc_scatter(64, src64, dst64, ew64, xs2, zacc64)[:N]

    batchf = batch.reshape(N, 1)
    y = _tc_final(acc2, xs2, dinv, b2.reshape(1, 64), batchf,
                  fc1_w, fc1_b.reshape(1, 128), fc2_w, fc2_b.reshape(1, 2))
    return y


# speedup vs baseline: 1.3167x; 1.3167x over previous
import functools

import jax
import jax.numpy as jnp
from jax import lax
from jax.experimental import pallas as pl
from jax.experimental.pallas import tpu as pltpu
from jax.experimental.pallas import tpu_sc as plsc

N = 10000
E = 320000
G = 64
NC, NS, LANES = 2, 16, 16
NW = NC * NS
GROUP = 128
DGPW = 160
GROUP64 = 64
AG0 = 224
AG1 = 96
CHG = 32
ZROWS = 64
NBUF = 4
E_PAD = NS * (AG0 + AG1) * GROUP64
NPAD = 10240
ROWS_PT = NPAD // NS
NDPAD = 10240
DEG_PT = NDPAD // NS

_f32 = jnp.float32

_MESH = plsc.VectorSubcoreMesh(
    core_axis_name="c", subcore_axis_name="s", num_cores=NC, num_subcores=NS)


def _worker_id():
    c = lax.axis_index("c")
    s = lax.axis_index("s")
    return c, s, c * NS + s


def _sc_deg_body(dst_hbm, ew_hbm, zdeg_hbm, out_hbm, dst_all, ew_all,
                 deg_sh, sem):
    c, s, _ = _worker_id()

    @pl.when(c == 0)
    def _work():
        pltpu.sync_copy(zdeg_hbm.at[pl.ds(s * DEG_PT, DEG_PT)],
                        deg_sh.at[pl.ds(s * DEG_PT, DEG_PT)])
        pltpu.sync_copy(dst_hbm.at[pl.ds(s * DGPW, DGPW), :], dst_all)
        pltpu.sync_copy(ew_hbm.at[pl.ds(s * DGPW, DGPW), :], ew_all)
        plsc.subcore_barrier()

        def chunk(t, _):
            for b in range(16):
                g = t * 16 + b
                pltpu.async_copy(ew_all.at[g], deg_sh.at[dst_all.at[g]], sem,
                                 add=True)
            for b in range(16):
                g = t * 16 + b
                pltpu.make_async_copy(ew_all.at[g], deg_sh.at[dst_all.at[g]],
                                      sem).wait()
            return 0

        lax.fori_loop(0, DGPW // 16, chunk, 0)
        plsc.subcore_barrier()
        pltpu.sync_copy(deg_sh.at[pl.ds(s * DEG_PT, DEG_PT)],
                        out_hbm.at[pl.ds(s * DEG_PT, DEG_PT)])


def _sc_deg(dst2d, ew2d, zdeg):
    k = pl.kernel(
        _sc_deg_body,
        out_type=jax.ShapeDtypeStruct((NDPAD,), _f32),
        mesh=_MESH,
        scratch_types=[
            pltpu.VMEM((DGPW, GROUP), jnp.int32),
            pltpu.VMEM((DGPW, GROUP), _f32),
            pltpu.VMEM_SHARED((NDPAD,), _f32),
            pltpu.SemaphoreType.DMA,
        ],
    )
    return k(dst2d, ew2d, zdeg)


def _sc_scatter_body(F, src_hbm, dst_hbm, ew_hbm, xs_hbm, out_hbm,
                     src_all, dst_all, ew_all, rows0, rows1, rows2, rows3,
                     zbuf, acc_sh, gsem0, gsem1, gsem2, gsem3,
                     ssem0, ssem1, ssem2, ssem3):
    c, s, _ = _worker_id()
    nfb = F // LANES
    rows = (rows0, rows1, rows2, rows3)
    gsem = (gsem0, gsem1, gsem2, gsem3)
    ssem = (ssem0, ssem1, ssem2, ssem3)

    gbase = jnp.where(c == 0, s * AG0, NS * AG0 + s * AG1)
    nch = jnp.where(c == 0, AG0 // CHG, AG1 // CHG)

    def chunk_loop(t, _):
        base = gbase + t * CHG
        pltpu.sync_copy(src_hbm.at[pl.ds(base, CHG), :], src_all)
        pltpu.sync_copy(dst_hbm.at[pl.ds(base, CHG), :], dst_all)
        pltpu.sync_copy(ew_hbm.at[pl.ds(base, CHG), :], ew_all)
        pltpu.async_copy(xs_hbm.at[src_all.at[0]], rows[0], gsem[0])
        pltpu.async_copy(xs_hbm.at[src_all.at[1]], rows[1], gsem[1])

        def outer(u, _):
            for b in range(NBUF):
                g = u * NBUF + b
                rows_b = rows[b]
                pltpu.make_async_copy(xs_hbm.at[src_all.at[g]], rows_b,
                                      gsem[b]).wait()

                for jg in range(GROUP64 // LANES):
                    wv = ew_all[g, pl.ds(jg * LANES, LANES)]
                    for l in range(LANES):
                        w = jnp.full((LANES,), wv[l], dtype=_f32)
                        j = jg * LANES + l
                        for f in range(nfb):
                            rows_b[j, pl.ds(f * LANES, LANES)] = (
                                rows_b[j, pl.ds(f * LANES, LANES)] * w)
                pltpu.async_copy(rows_b, acc_sh.at[dst_all.at[g]], ssem[b],
                                 add=True)

                bn = (b + 2) % NBUF
                rows_n = rows[bn]

                @pl.when(g + 2 < CHG)
                def _next():
                    @pl.when(g >= 2)
                    def _drain():
                        pltpu.make_async_copy(
                            rows_n, acc_sh.at[dst_all.at[g]], ssem[bn]).wait()

                    pltpu.async_copy(xs_hbm.at[src_all.at[g + 2]], rows_n,
                                     gsem[bn])
            return 0

        lax.fori_loop(0, CHG // NBUF, outer, 0)
        for b in range(NBUF):
            pltpu.make_async_copy(rows[b], acc_sh.at[dst_all.at[0]],
                                  ssem[b]).wait()
        return 0

    zv = jnp.zeros((LANES,), _f32)
    for r in range(ZROWS):
        for f in range(nfb):
            zbuf[r, pl.ds(f * LANES, LANES)] = zv
    for i in range(ROWS_PT // ZROWS):
        pltpu.sync_copy(zbuf,
                        acc_sh.at[pl.ds(s * ROWS_PT + i * ZROWS, ZROWS), :])
    plsc.subcore_barrier()
    lax.fori_loop(0, nch, chunk_loop, 0)
    plsc.subcore_barrier()
    pltpu.sync_copy(acc_sh.at[pl.ds(s * ROWS_PT, ROWS_PT), :],
                    out_hbm.at[pl.ds(c * NPAD + s * ROWS_PT, ROWS_PT), :])


def _sc_scatter(F, src2d, dst2d, ew2d, xs):
    k = pl.kernel(
        functools.partial(_sc_scatter_body, F),
        out_type=jax.ShapeDtypeStruct((NC * NPAD, F), _f32),
        mesh=_MESH,
        scratch_types=(
            [pltpu.VMEM((CHG, GROUP64), jnp.int32),
             pltpu.VMEM((CHG, GROUP64), jnp.int32),
             pltpu.VMEM((CHG, GROUP64), _f32)]
            + [pltpu.VMEM((GROUP64, F), _f32) for _ in range(NBUF)]
            + [pltpu.VMEM((ZROWS, F), _f32)]
            + [pltpu.VMEM_SHARED((NPAD, F), _f32)]
            + [pltpu.SemaphoreType.DMA for _ in range(2 * NBUF)]
        ),
        compiler_params=pltpu.CompilerParams(use_tc_tiling_on_sc=False),
    )
    return k(src2d, dst2d, ew2d, xs)


_BLK = 1000
_NBLK = N // _BLK


def _tc_prep1_body(x_ref, degt_ref, w1_ref, xs_ref, dinv_ref):
    deg = jnp.sum(degt_ref[...], axis=1, keepdims=True) + 1.0
    dinv = lax.rsqrt(deg)
    dinv_ref[...] = dinv
    xs_ref[...] = jnp.dot(x_ref[...], w1_ref[...],
                          preferred_element_type=_f32) * dinv


def _tc_prep1(x, degt, W1):
    return pl.pallas_call(
        _tc_prep1_body,
        grid=(_NBLK,),
        in_specs=[
            pl.BlockSpec((_BLK, 128), lambda i: (i, 0)),
            pl.BlockSpec((_BLK, 1), lambda i: (i, 0)),
            pl.BlockSpec((128, 128), lambda i: (0, 0)),
        ],
        out_specs=[
            pl.BlockSpec((_BLK, 128), lambda i: (i, 0)),
            pl.BlockSpec((_BLK, 1), lambda i: (i, 0)),
        ],
        out_shape=[
            jax.ShapeDtypeStruct((N, 128), _f32),
            jax.ShapeDtypeStruct((N, 1), _f32),
        ],
    )(x, degt, W1)


def _tc_mid_body(acc_ref, xs1_ref, dinv_ref, w2_ref, b1_ref, xs2_ref):
    dinv = dinv_ref[...]
    agg = acc_ref[0] + acc_ref[1] + xs1_ref[...]
    h = jax.nn.relu(dinv * agg + b1_ref[...])
    xs2_ref[...] = jnp.dot(h, w2_ref[...], preferred_element_type=_f32) * dinv


def _tc_mid(acc1, xs1, dinv, W2, b1):
    return pl.pallas_call(
        _tc_mid_body,
        grid=(_NBLK,),
        in_specs=[
            pl.BlockSpec((NC, _BLK, 128), lambda i: (0, i, 0)),
            pl.BlockSpec((_BLK, 128), lambda i: (i, 0)),
            pl.BlockSpec((_BLK, 1), lambda i: (i, 0)),
            pl.BlockSpec((128, 64), lambda i: (0, 0)),
            pl.BlockSpec((1, 128), lambda i: (0, 0)),
        ],
        out_specs=pl.BlockSpec((_BLK, 64), lambda i: (i, 0)),
        out_shape=jax.ShapeDtypeStruct((N, 64), _f32),
    )(acc1, xs1, dinv, W2, b1)


def _tc_final_body(acc_ref, xs2_ref, dinv_ref, b2_ref, batch_ref,
                   fc1w_ref, fc1b_ref, fc2w_ref, fc2b_ref, y_ref,
                   pool_acc, cnt_acc):
    i = pl.program_id(0)
    agg = acc_ref[0] + acc_ref[1] + xs2_ref[...]
    x1 = jax.nn.relu(dinv_ref[...] * agg + b2_ref[...])
    gids = lax.broadcasted_iota(jnp.int32, (_BLK, G), 1)
    onehot = jnp.where(batch_ref[...] == gids, 1.0, 0.0)
    pool_blk = lax.dot_general(onehot, x1, (((0,), (0,)), ((), ())),
                               preferred_element_type=_f32)
    cnt_blk = lax.dot_general(onehot, jnp.ones((_BLK, 1), _f32),
                              (((0,), (0,)), ((), ())),
                              preferred_element_type=_f32)

    @pl.when(i == 0)
    def _init():
        pool_acc[...] = jnp.zeros_like(pool_acc)
        cnt_acc[...] = jnp.zeros_like(cnt_acc)

    pool_acc[...] += pool_blk
    cnt_acc[...] += cnt_blk

    @pl.when(i == _NBLK - 1)
    def _fin():
        mean = pool_acc[...] / jnp.maximum(cnt_acc[...], 1.0)
        h2 = jax.nn.relu(jnp.dot(mean, fc1w_ref[...],
                                 preferred_element_type=_f32) + fc1b_ref[...])
        logits = jnp.dot(h2, fc2w_ref[...],
                         preferred_element_type=_f32) + fc2b_ref[...]
        m = jnp.max(logits, axis=1, keepdims=True)
        lse = m + jnp.log(jnp.sum(jnp.exp(logits - m), axis=1, keepdims=True))
        y_ref[...] = logits - lse


def _tc_final(acc2, xs2, dinv, b2, batchf, fc1_w, fc1_b, fc2_w, fc2_b):
    return pl.pallas_call(
        _tc_final_body,
        grid=(_NBLK,),
        in_specs=[
            pl.BlockSpec((NC, _BLK, 64), lambda i: (0, i, 0)),
            pl.BlockSpec((_BLK, 64), lambda i: (i, 0)),
            pl.BlockSpec((_BLK, 1), lambda i: (i, 0)),
            pl.BlockSpec((1, 64), lambda i: (0, 0)),
            pl.BlockSpec((_BLK, 1), lambda i: (i, 0)),
            pl.BlockSpec((64, 128), lambda i: (0, 0)),
            pl.BlockSpec((1, 128), lambda i: (0, 0)),
            pl.BlockSpec((128, 2), lambda i: (0, 0)),
            pl.BlockSpec((1, 2), lambda i: (0, 0)),
        ],
        out_specs=pl.BlockSpec((G, 2), lambda i: (0, 0)),
        out_shape=jax.ShapeDtypeStruct((G, 2), _f32),
        scratch_shapes=[
            pltpu.VMEM((G, 64), _f32),
            pltpu.VMEM((G, 1), _f32),
        ],
        compiler_params=pltpu.CompilerParams(
            dimension_semantics=("arbitrary",)),
    )(acc2, xs2, dinv, b2, batchf, fc1_w, fc1_b, fc2_w, fc2_b)


def kernel(x, edge_index, edge_attr, batch, W1, b1, W2, b2,
           fc1_w, fc1_b, fc2_w, fc2_b):
    ew = jnp.squeeze(edge_attr).astype(_f32)
    src = edge_index[0]
    dst = edge_index[1]

    pad = E_PAD - E
    srcp = jnp.concatenate([src, jnp.zeros((pad,), src.dtype)])
    dstp = jnp.concatenate([dst, jnp.zeros((pad,), dst.dtype)])
    ewp = jnp.concatenate([ew, jnp.zeros((pad,), _f32)])
    dst2d = dstp.reshape(-1, GROUP)
    ew2d = ewp.reshape(-1, GROUP)
    src64 = srcp.reshape(-1, GROUP64)
    dst64 = dstp.reshape(-1, GROUP64)
    ew64 = ewp.reshape(-1, GROUP64)

    zdeg = jnp.zeros((NDPAD,), _f32)

    degp = _sc_deg(dst2d, ew2d, zdeg)
    degt = degp[:N].reshape(N, 1)

    xs1, dinv = _tc_prep1(x, degt, W1)
    acc1 = _sc_scatter(128, src64, dst64, ew64, xs1).reshape(NC, NPAD, 128)[:, :N]
    xs2 = _tc_mid(acc1, xs1, dinv, W2, b1.reshape(1, 128))
    acc2 = _sc_scatter(64, src64, dst64, ew64, xs2).reshape(NC, NPAD, 64)[:, :N]

    batchf = batch.reshape(N, 1)
    y = _tc_final(acc2, xs2, dinv, b2.reshape(1, 64), batchf,
                  fc1_w, fc1_b.reshape(1, 128), fc2_w, fc2_b.reshape(1, 2))
    return y


# speedup vs baseline: 1.3278x; 1.0085x over previous
import functools

import jax
import jax.numpy as jnp
from jax import lax
from jax.experimental import pallas as pl
from jax.experimental.pallas import tpu as pltpu
from jax.experimental.pallas import tpu_sc as plsc

N = 10000
E = 320000
G = 64
NC, NS, LANES = 2, 16, 16
NW = NC * NS
GROUP = 128
DGPW = 160
GROUP64 = 64
AG0 = 224
AG1 = 96
CHG = 32
ZROWS = 64
NBUF = 4
E_PAD = NS * (AG0 + AG1) * GROUP64
NPAD = 10240
ROWS_PT = NPAD // NS
NDPAD = 10240
DEG_PT = NDPAD // NS

_f32 = jnp.float32

_MESH = plsc.VectorSubcoreMesh(
    core_axis_name="c", subcore_axis_name="s", num_cores=NC, num_subcores=NS)


def _worker_id():
    c = lax.axis_index("c")
    s = lax.axis_index("s")
    return c, s, c * NS + s


def _sc_deg_body(dst_hbm, ew_hbm, zdeg_hbm, out_hbm, dst_all, ew_all,
                 deg_sh, sem):
    c, s, _ = _worker_id()

    @pl.when(c == 0)
    def _work():
        pltpu.sync_copy(zdeg_hbm.at[pl.ds(s * DEG_PT, DEG_PT)],
                        deg_sh.at[pl.ds(s * DEG_PT, DEG_PT)])
        pltpu.sync_copy(dst_hbm.at[pl.ds(s * DGPW, DGPW), :], dst_all)
        pltpu.sync_copy(ew_hbm.at[pl.ds(s * DGPW, DGPW), :], ew_all)
        plsc.subcore_barrier()

        def chunk(t, _):
            for b in range(16):
                g = t * 16 + b
                pltpu.async_copy(ew_all.at[g], deg_sh.at[dst_all.at[g]], sem,
                                 add=True)
            for b in range(16):
                g = t * 16 + b
                pltpu.make_async_copy(ew_all.at[g], deg_sh.at[dst_all.at[g]],
                                      sem).wait()
            return 0

        lax.fori_loop(0, DGPW // 16, chunk, 0)
        plsc.subcore_barrier()
        pltpu.sync_copy(deg_sh.at[pl.ds(s * DEG_PT, DEG_PT)],
                        out_hbm.at[pl.ds(s * DEG_PT, DEG_PT)])


def _sc_deg(dst2d, ew2d, zdeg):
    k = pl.kernel(
        _sc_deg_body,
        out_type=jax.ShapeDtypeStruct((NDPAD,), _f32),
        mesh=_MESH,
        scratch_types=[
            pltpu.VMEM((DGPW, GROUP), jnp.int32),
            pltpu.VMEM((DGPW, GROUP), _f32),
            pltpu.VMEM_SHARED((NDPAD,), _f32),
            pltpu.SemaphoreType.DMA,
        ],
    )
    return k(dst2d, ew2d, zdeg)


def _sc_scatter_body(F, src_hbm, dst_hbm, ew_hbm, xs_hbm, out_hbm,
                     src_all, dst_all, ew_all, rows0, rows1, rows2, rows3,
                     zbuf, acc_sh, gsem0, gsem1, gsem2, gsem3,
                     ssem0, ssem1, ssem2, ssem3):
    c, s, _ = _worker_id()
    nfb = F // LANES
    rows = (rows0, rows1, rows2, rows3)
    gsem = (gsem0, gsem1, gsem2, gsem3)
    ssem = (ssem0, ssem1, ssem2, ssem3)

    gbase = jnp.where(c == 0, s * AG0, NS * AG0 + s * AG1)
    nch = jnp.where(c == 0, AG0 // CHG, AG1 // CHG)

    def chunk_loop(t, _):
        base = gbase + t * CHG
        pltpu.sync_copy(src_hbm.at[pl.ds(base, CHG), :], src_all)
        pltpu.sync_copy(dst_hbm.at[pl.ds(base, CHG), :], dst_all)
        pltpu.sync_copy(ew_hbm.at[pl.ds(base, CHG), :], ew_all)
        pltpu.async_copy(xs_hbm.at[src_all.at[0]], rows[0], gsem[0])
        pltpu.async_copy(xs_hbm.at[src_all.at[1]], rows[1], gsem[1])

        def outer(u, _):
            for b in range(NBUF):
                g = u * NBUF + b
                rows_b = rows[b]
                pltpu.make_async_copy(xs_hbm.at[src_all.at[g]], rows_b,
                                      gsem[b]).wait()

                for jg in range(GROUP64 // LANES):
                    wv = ew_all[g, pl.ds(jg * LANES, LANES)]
                    for l in range(LANES):
                        w = jnp.full((LANES,), wv[l], dtype=_f32)
                        j = jg * LANES + l
                        for f in range(nfb):
                            rows_b[j, pl.ds(f * LANES, LANES)] = (
                                rows_b[j, pl.ds(f * LANES, LANES)] * w)
                pltpu.async_copy(rows_b, acc_sh.at[dst_all.at[g]], ssem[b],
                                 add=True)

                bn = (b + 2) % NBUF
                rows_n = rows[bn]

                @pl.when(g + 2 < CHG)
                def _next():
                    @pl.when(g >= 2)
                    def _drain():
                        pltpu.make_async_copy(
                            rows_n, acc_sh.at[dst_all.at[g]], ssem[bn]).wait()

                    pltpu.async_copy(xs_hbm.at[src_all.at[g + 2]], rows_n,
                                     gsem[bn])
            return 0

        lax.fori_loop(0, CHG // NBUF, outer, 0)
        for b in range(NBUF):
            pltpu.make_async_copy(rows[b], acc_sh.at[dst_all.at[0]],
                                  ssem[b]).wait()
        return 0

    zv = jnp.zeros((LANES,), _f32)
    for r in range(ZROWS):
        for f in range(nfb):
            zbuf[r, pl.ds(f * LANES, LANES)] = zv
    for i in range(ROWS_PT // ZROWS):
        pltpu.sync_copy(zbuf,
                        acc_sh.at[pl.ds(s * ROWS_PT + i * ZROWS, ZROWS), :])
    plsc.subcore_barrier()
    lax.fori_loop(0, nch, chunk_loop, 0)
    plsc.subcore_barrier()
    pltpu.sync_copy(acc_sh.at[pl.ds(s * ROWS_PT, ROWS_PT), :],
                    out_hbm.at[pl.ds(c * NPAD + s * ROWS_PT, ROWS_PT), :])


def _sc_scatter(F, src2d, dst2d, ew2d, xs):
    k = pl.kernel(
        functools.partial(_sc_scatter_body, F),
        out_type=jax.ShapeDtypeStruct((NC * NPAD, F), _f32),
        mesh=_MESH,
        scratch_types=(
            [pltpu.VMEM((CHG, GROUP64), jnp.int32),
             pltpu.VMEM((CHG, GROUP64), jnp.int32),
             pltpu.VMEM((CHG, GROUP64), _f32)]
            + [pltpu.VMEM((GROUP64, F), _f32) for _ in range(NBUF)]
            + [pltpu.VMEM((ZROWS, F), _f32)]
            + [pltpu.VMEM_SHARED((NPAD, F), _f32)]
            + [pltpu.SemaphoreType.DMA for _ in range(2 * NBUF)]
        ),
        compiler_params=pltpu.CompilerParams(use_tc_tiling_on_sc=False),
    )
    return k(src2d, dst2d, ew2d, xs)


_BLK = 2000
_NBLK = N // _BLK


def _tc_prep1_body(x_ref, degt_ref, w1_ref, xs_ref, dinv_ref):
    deg = jnp.sum(degt_ref[...], axis=1, keepdims=True) + 1.0
    dinv = lax.rsqrt(deg)
    dinv_ref[...] = dinv
    xs_ref[...] = jnp.dot(x_ref[...], w1_ref[...],
                          preferred_element_type=_f32) * dinv


def _tc_prep1(x, degt, W1):
    return pl.pallas_call(
        _tc_prep1_body,
        grid=(_NBLK,),
        in_specs=[
            pl.BlockSpec((_BLK, 128), lambda i: (i, 0)),
            pl.BlockSpec((_BLK, 1), lambda i: (i, 0)),
            pl.BlockSpec((128, 128), lambda i: (0, 0)),
        ],
        out_specs=[
            pl.BlockSpec((_BLK, 128), lambda i: (i, 0)),
            pl.BlockSpec((_BLK, 1), lambda i: (i, 0)),
        ],
        out_shape=[
            jax.ShapeDtypeStruct((N, 128), _f32),
            jax.ShapeDtypeStruct((N, 1), _f32),
        ],
    )(x, degt, W1)


def _tc_mid_body(acc_ref, xs1_ref, dinv_ref, w2_ref, b1_ref, xs2_ref):
    dinv = dinv_ref[...]
    agg = acc_ref[0] + acc_ref[1] + xs1_ref[...]
    h = jax.nn.relu(dinv * agg + b1_ref[...])
    xs2_ref[...] = jnp.dot(h, w2_ref[...], preferred_element_type=_f32) * dinv


def _tc_mid(acc1, xs1, dinv, W2, b1):
    return pl.pallas_call(
        _tc_mid_body,
        grid=(_NBLK,),
        in_specs=[
            pl.BlockSpec((NC, _BLK, 128), lambda i: (0, i, 0)),
            pl.BlockSpec((_BLK, 128), lambda i: (i, 0)),
            pl.BlockSpec((_BLK, 1), lambda i: (i, 0)),
            pl.BlockSpec((128, 64), lambda i: (0, 0)),
            pl.BlockSpec((1, 128), lambda i: (0, 0)),
        ],
        out_specs=pl.BlockSpec((_BLK, 64), lambda i: (i, 0)),
        out_shape=jax.ShapeDtypeStruct((N, 64), _f32),
    )(acc1, xs1, dinv, W2, b1)


def _tc_final_body(acc_ref, xs2_ref, dinv_ref, b2_ref, batch_ref,
                   fc1w_ref, fc1b_ref, fc2w_ref, fc2b_ref, y_ref,
                   pool_acc, cnt_acc):
    i = pl.program_id(0)
    agg = acc_ref[0] + acc_ref[1] + xs2_ref[...]
    x1 = jax.nn.relu(dinv_ref[...] * agg + b2_ref[...])
    gids = lax.broadcasted_iota(jnp.int32, (_BLK, G), 1)
    onehot = jnp.where(batch_ref[...] == gids, 1.0, 0.0)
    pool_blk = lax.dot_general(onehot, x1, (((0,), (0,)), ((), ())),
                               preferred_element_type=_f32)
    cnt_blk = lax.dot_general(onehot, jnp.ones((_BLK, 1), _f32),
                              (((0,), (0,)), ((), ())),
                              preferred_element_type=_f32)

    @pl.when(i == 0)
    def _init():
        pool_acc[...] = jnp.zeros_like(pool_acc)
        cnt_acc[...] = jnp.zeros_like(cnt_acc)

    pool_acc[...] += pool_blk
    cnt_acc[...] += cnt_blk

    @pl.when(i == _NBLK - 1)
    def _fin():
        mean = pool_acc[...] / jnp.maximum(cnt_acc[...], 1.0)
        h2 = jax.nn.relu(jnp.dot(mean, fc1w_ref[...],
                                 preferred_element_type=_f32) + fc1b_ref[...])
        logits = jnp.dot(h2, fc2w_ref[...],
                         preferred_element_type=_f32) + fc2b_ref[...]
        m = jnp.max(logits, axis=1, keepdims=True)
        lse = m + jnp.log(jnp.sum(jnp.exp(logits - m), axis=1, keepdims=True))
        y_ref[...] = logits - lse


def _tc_final(acc2, xs2, dinv, b2, batchf, fc1_w, fc1_b, fc2_w, fc2_b):
    return pl.pallas_call(
        _tc_final_body,
        grid=(_NBLK,),
        in_specs=[
            pl.BlockSpec((NC, _BLK, 64), lambda i: (0, i, 0)),
            pl.BlockSpec((_BLK, 64), lambda i: (i, 0)),
            pl.BlockSpec((_BLK, 1), lambda i: (i, 0)),
            pl.BlockSpec((1, 64), lambda i: (0, 0)),
            pl.BlockSpec((_BLK, 1), lambda i: (i, 0)),
            pl.BlockSpec((64, 128), lambda i: (0, 0)),
            pl.BlockSpec((1, 128), lambda i: (0, 0)),
            pl.BlockSpec((128, 2), lambda i: (0, 0)),
            pl.BlockSpec((1, 2), lambda i: (0, 0)),
        ],
        out_specs=pl.BlockSpec((G, 2), lambda i: (0, 0)),
        out_shape=jax.ShapeDtypeStruct((G, 2), _f32),
        scratch_shapes=[
            pltpu.VMEM((G, 64), _f32),
            pltpu.VMEM((G, 1), _f32),
        ],
        compiler_params=pltpu.CompilerParams(
            dimension_semantics=("arbitrary",)),
    )(acc2, xs2, dinv, b2, batchf, fc1_w, fc1_b, fc2_w, fc2_b)


def kernel(x, edge_index, edge_attr, batch, W1, b1, W2, b2,
           fc1_w, fc1_b, fc2_w, fc2_b):
    ew = jnp.squeeze(edge_attr).astype(_f32)
    src = edge_index[0]
    dst = edge_index[1]

    pad = E_PAD - E
    srcp = jnp.concatenate([src, jnp.zeros((pad,), src.dtype)])
    dstp = jnp.concatenate([dst, jnp.zeros((pad,), dst.dtype)])
    ewp = jnp.concatenate([ew, jnp.zeros((pad,), _f32)])
    dst2d = dstp.reshape(-1, GROUP)
    ew2d = ewp.reshape(-1, GROUP)
    src64 = srcp.reshape(-1, GROUP64)
    dst64 = dstp.reshape(-1, GROUP64)
    ew64 = ewp.reshape(-1, GROUP64)

    zdeg = jnp.zeros((NDPAD,), _f32)

    degp = _sc_deg(dst2d, ew2d, zdeg)
    degt = degp[:N].reshape(N, 1)

    xs1, dinv = _tc_prep1(x, degt, W1)
    acc1 = _sc_scatter(128, src64, dst64, ew64, xs1).reshape(NC, NPAD, 128)[:, :N]
    xs2 = _tc_mid(acc1, xs1, dinv, W2, b1.reshape(1, 128))
    acc2 = _sc_scatter(64, src64, dst64, ew64, xs2).reshape(NC, NPAD, 64)[:, :N]

    batchf = batch.reshape(N, 1)
    y = _tc_final(acc2, xs2, dinv, b2.reshape(1, 64), batchf,
                  fc1_w, fc1_b.reshape(1, 128), fc2_w, fc2_b.reshape(1, 2))
    return y
